# baseline (speedup 1.0000x reference)
import functools

import jax
import jax.numpy as jnp
from jax import lax
from jax.experimental import pallas as pl
from jax.experimental.pallas import tpu as pltpu
from jax.experimental.pallas import tpu_sc as plsc

_B, _C, _N, _K = 8, 3, 2048, 32
_O1, _O2 = 64, 64
_R = 256
_QS = _N // 4
_EPS = 1e-5


def _knn_body(xrow_ref, xall_ref, idx_ref, nd_ref):
    xr = xrow_ref[0]
    xa = xall_ref[0]
    xxr = jnp.sum(xr * xr, axis=0)
    xxa = jnp.sum(xa * xa, axis=0)
    inner = -2.0 * lax.dot_general(
        xr, xa, (((0,), (0,)), ((), ())),
        preferred_element_type=jnp.float32)
    nd_ref[...] = (-xxr[:, None]) - inner - xxa[None, :]

    col = lax.broadcasted_iota(jnp.int32, (_R, _N), 1)
    kcol = lax.broadcasted_iota(jnp.int32, (_R, _K), 1)

    def step(j, acc):
        nd = nd_ref[...]
        amin = jnp.argmax(nd, axis=1, keepdims=True).astype(jnp.int32)
        nd_ref[...] = jnp.where(col == amin, -jnp.inf, nd)
        return jnp.where(kcol == j, amin, acc)

    idx_ref[0] = lax.fori_loop(0, _K, step, jnp.zeros((_R, _K), jnp.int32))


def _knn(x, interpret=False):
    return pl.pallas_call(
        _knn_body,
        grid=(_B, _N // _R),
        in_specs=[
            pl.BlockSpec((1, _C, _R), lambda b, nb: (b, 0, nb)),
            pl.BlockSpec((1, _C, _N), lambda b, nb: (b, 0, 0)),
        ],
        out_specs=pl.BlockSpec((1, _R, _K), lambda b, nb: (b, nb, 0)),
        out_shape=jax.ShapeDtypeStruct((_B, _N, _K), jnp.int32),
        scratch_shapes=[pltpu.VMEM((_R, _N), jnp.float32)],
        interpret=interpret,
    )(x, x)


def _gather(x, idx):
    info = plsc.get_sparse_core_info()
    ch = _QS * _K
    outw = _C * _K * _QS
    xflat = x.reshape(_B, _C * _N)
    idxflat = idx.reshape(_B, _N * _K)

    mesh = plsc.VectorSubcoreMesh(core_axis_name="c", subcore_axis_name="s")

    @functools.partial(
        pl.kernel, mesh=mesh,
        compiler_params=pltpu.CompilerParams(needs_layout_passes=False),
        out_type=jax.ShapeDtypeStruct((_B, 4 * outw), jnp.float32),
        scratch_types=[
            pltpu.VMEM((_C * _N,), jnp.float32),
            pltpu.VMEM((ch,), jnp.int32),
            pltpu.VMEM((outw,), jnp.float32),
        ],
    )
    def k(x_hbm, idx_hbm, out_hbm, xv, iv, ov):
        wid = lax.axis_index("s") * info.num_cores + lax.axis_index("c")
        b = wid // 4
        q = wid % 4
        pltpu.sync_copy(x_hbm.at[b], xv)
        pltpu.sync_copy(idx_hbm.at[b, pl.ds(q * ch, ch)], iv)
        lanes = lax.iota(jnp.int32, 16)
        n0 = q * _QS

        def per_k(kk, carry):
            for i in range(_QS // 16):
                pos = i * 16 + lanes
                nbr = plsc.load_gather(iv, [pos * _K + kk])
                for c in range(_C):
                    cbase = c * _N
                    nv = plsc.load_gather(xv, [nbr + cbase])
                    cv = plsc.load_gather(xv, [pos + (n0 + cbase)])
                    off = (c * _K + kk) * _QS + i * 16
                    plsc.store_scatter(ov, [off + lanes], nv - cv)
            return carry

        lax.fori_loop(0, _K, per_k, 0)
        pltpu.sync_copy(ov, out_hbm.at[b, pl.ds(q * outw, outw)])

    return k(xflat, idxflat).reshape(_B, 4, _C, _K, _QS)


def _z1_of(xrow_ref, d_ref, w1a_ref, w1b_ref):
    xr = xrow_ref[0]
    d = d_ref[0, 0]
    z1c = lax.dot_general(w1a_ref[...], xr, (((1,), (0,)), ((), ())),
                          preferred_element_type=jnp.float32)
    z1d = lax.dot_general(w1b_ref[...], d.reshape(_C, _K * _R),
                          (((1,), (0,)), ((), ())),
                          preferred_element_type=jnp.float32)
    return z1d.reshape(_O1, _K, _R) + z1c[:, None, :]


def _lrelu(v):
    return jnp.where(v >= 0, v, 0.2 * v)


_CNT = float(_B * _N * _K)


def _mlp_body(xrow_ref, d_ref, w1a_ref, w1b_ref, g1_ref, be1_ref, w2_ref,
              g2_ref, be2_ref, out_ref, s1_ref, q1_ref, s2_ref, q2_ref,
              ab1_ref, ab2_ref):
    p = pl.program_id(0)
    first = (pl.program_id(1) == 0) & (pl.program_id(2) == 0)

    def accum(val2d, s_ref, q_ref):
        @pl.when(first)
        def _():
            s_ref[...] = jnp.zeros_like(s_ref)
            q_ref[...] = jnp.zeros_like(q_ref)
        s_ref[...] += jnp.sum(val2d, axis=1)[None, :]
        q_ref[...] += jnp.sum(val2d * val2d, axis=1)[None, :]

    def finalize(s_ref, q_ref, g_ref, be_ref, ab_ref):
        m = s_ref[...] / _CNT
        v = q_ref[...] / _CNT - m * m
        a = g_ref[...] * lax.rsqrt(v + _EPS)
        ab_ref[0:1, :] = a
        ab_ref[1:2, :] = be_ref[...] - a * m

    def z2_of():
        z1 = _z1_of(xrow_ref, d_ref, w1a_ref, w1b_ref)
        a1 = ab1_ref[0]
        b1 = ab1_ref[1]
        y1 = _lrelu(z1 * a1[:, None, None] + b1[:, None, None])
        return lax.dot_general(w2_ref[...], y1.reshape(_O1, _K * _R),
                               (((1,), (0,)), ((), ())),
                               preferred_element_type=jnp.float32)

    @pl.when(p == 0)
    def _():
        z1 = _z1_of(xrow_ref, d_ref, w1a_ref, w1b_ref)
        accum(z1.reshape(_O1, _K * _R), s1_ref, q1_ref)

    @pl.when(p == 1)
    def _():
        @pl.when(first)
        def _():
            finalize(s1_ref, q1_ref, g1_ref, be1_ref, ab1_ref)
        accum(z2_of(), s2_ref, q2_ref)

    @pl.when(p == 2)
    def _():
        @pl.when(first)
        def _():
            finalize(s2_ref, q2_ref, g2_ref, be2_ref, ab2_ref)
        z2 = z2_of()
        a2 = ab2_ref[0]
        b2 = ab2_ref[1]
        y2 = _lrelu(z2 * a2[:, None] + b2[:, None]).reshape(_O2, _K, _R)
        m = y2[:, 0, :]
        for kk in range(1, _K):
            m = jnp.maximum(m, y2[:, kk, :])
        out_ref[0] = m


def _mlp(x, d5, w1a, w1b, g1, be1, w2, g2, be2, interpret=False):
    vec = pl.BlockSpec((1, _O1), lambda p, b, nb: (0, 0))
    wspec = pl.BlockSpec((_O1, _C), lambda p, b, nb: (0, 0))
    return pl.pallas_call(
        _mlp_body,
        grid=(3, _B, _N // _R),
        in_specs=[
            pl.BlockSpec((1, _C, _R), lambda p, b, nb: (b, 0, nb)),
            pl.BlockSpec((1, 1, _C, _K, _R),
                         lambda p, b, nb: (b, nb // 2, 0, 0, nb % 2)),
            wspec, wspec, vec, vec,
            pl.BlockSpec((_O2, _O1), lambda p, b, nb: (0, 0)),
            vec, vec,
        ],
        out_specs=pl.BlockSpec((1, _O2, _R), lambda p, b, nb: (b, 0, nb)),
        out_shape=jax.ShapeDtypeStruct((_B, _O2, _N), jnp.float32),
        scratch_shapes=[
            pltpu.VMEM((1, _O1), jnp.float32),
            pltpu.VMEM((1, _O1), jnp.float32),
            pltpu.VMEM((1, _O2), jnp.float32),
            pltpu.VMEM((1, _O2), jnp.float32),
            pltpu.VMEM((2, _O1), jnp.float32),
            pltpu.VMEM((2, _O2), jnp.float32),
        ],
        interpret=interpret,
    )(x, d5, w1a, w1b, g1, be1, w2, g2, be2)


def kernel(x, W1, gamma1, beta1, W2, gamma2, beta2):
    idx = _knn(x)
    d5 = _gather(x, idx)
    return _mlp(x, d5, W1[:, :_C], W1[:, _C:], gamma1[None, :],
                beta1[None, :], W2, gamma2[None, :], beta2[None, :])


# speedup vs baseline: 1.1545x; 1.1545x over previous
import functools

import jax
import jax.numpy as jnp
from jax import lax
from jax.experimental import pallas as pl
from jax.experimental.pallas import tpu as pltpu
from jax.experimental.pallas import tpu_sc as plsc

_B, _C, _N, _K = 8, 3, 2048, 32
_O1, _O2 = 64, 64
_R = 256
_QS = _N // 4
_EPS = 1e-5


def _knn_body(xrow_ref, xall_ref, idx_ref, nd_ref):
    xr = xrow_ref[0]
    xa = xall_ref[0]
    xxr = jnp.sum(xr * xr, axis=0)
    xxa = jnp.sum(xa * xa, axis=0)
    inner = -2.0 * lax.dot_general(
        xr, xa, (((0,), (0,)), ((), ())),
        preferred_element_type=jnp.float32)
    nd_ref[...] = (-xxr[:, None]) - inner - xxa[None, :]

    col = lax.broadcasted_iota(jnp.int32, (_R, _N), 1)
    kcol = lax.broadcasted_iota(jnp.int32, (_R, _K), 1)

    def step(j, acc):
        nd = nd_ref[...]
        m1 = jnp.max(nd, axis=1, keepdims=True)
        a1 = jnp.min(jnp.where(nd == m1, col, _N), axis=1, keepdims=True)
        nd2 = jnp.where(col == a1, -jnp.inf, nd)
        m2 = jnp.max(nd2, axis=1, keepdims=True)
        a2 = jnp.min(jnp.where(nd2 == m2, col, _N), axis=1, keepdims=True)
        nd_ref[...] = jnp.where(col == a2, -jnp.inf, nd2)
        acc = jnp.where(kcol == 2 * j, a1, acc)
        return jnp.where(kcol == 2 * j + 1, a2, acc)

    idx_ref[0] = lax.fori_loop(0, _K // 2, step,
                               jnp.zeros((_R, _K), jnp.int32))


def _knn(x, interpret=False):
    return pl.pallas_call(
        _knn_body,
        grid=(_B, _N // _R),
        in_specs=[
            pl.BlockSpec((1, _C, _R), lambda b, nb: (b, 0, nb)),
            pl.BlockSpec((1, _C, _N), lambda b, nb: (b, 0, 0)),
        ],
        out_specs=pl.BlockSpec((1, _R, _K), lambda b, nb: (b, nb, 0)),
        out_shape=jax.ShapeDtypeStruct((_B, _N, _K), jnp.int32),
        scratch_shapes=[pltpu.VMEM((_R, _N), jnp.float32)],
        interpret=interpret,
    )(x, x)


def _gather(x, idx):
    info = plsc.get_sparse_core_info()
    ch = _QS * _K
    outw = _C * _K * _QS
    xflat = x.reshape(_B, _C * _N)
    idxflat = idx.reshape(_B, _N * _K)

    mesh = plsc.VectorSubcoreMesh(core_axis_name="c", subcore_axis_name="s")

    @functools.partial(
        pl.kernel, mesh=mesh,
        compiler_params=pltpu.CompilerParams(needs_layout_passes=False),
        out_type=jax.ShapeDtypeStruct((_B, 4 * outw), jnp.float32),
        scratch_types=[
            pltpu.VMEM((_C * _N,), jnp.float32),
            pltpu.VMEM((ch,), jnp.int32),
            pltpu.VMEM((outw,), jnp.float32),
        ],
    )
    def k(x_hbm, idx_hbm, out_hbm, xv, iv, ov):
        wid = lax.axis_index("s") * info.num_cores + lax.axis_index("c")
        b = wid // 4
        q = wid % 4
        pltpu.sync_copy(x_hbm.at[b], xv)
        pltpu.sync_copy(idx_hbm.at[b, pl.ds(q * ch, ch)], iv)
        lanes = lax.iota(jnp.int32, 16)
        n0 = q * _QS

        def per_k(kk, carry):
            for i in range(_QS // 16):
                pos = i * 16 + lanes
                nbr = plsc.load_gather(iv, [pos * _K + kk])
                for c in range(_C):
                    cbase = c * _N
                    nv = plsc.load_gather(xv, [nbr + cbase])
                    cv = plsc.load_gather(xv, [pos + (n0 + cbase)])
                    off = (c * _K + kk) * _QS + i * 16
                    plsc.store_scatter(ov, [off + lanes], nv - cv)
            return carry

        lax.fori_loop(0, _K, per_k, 0)
        pltpu.sync_copy(ov, out_hbm.at[b, pl.ds(q * outw, outw)])

    return k(xflat, idxflat).reshape(_B, 4, _C, _K, _QS)


def _z1_of(xrow_ref, d_ref, w1a_ref, w1b_ref):
    xr = xrow_ref[0]
    d = d_ref[0, 0]
    z1c = lax.dot_general(w1a_ref[...], xr, (((1,), (0,)), ((), ())),
                          preferred_element_type=jnp.float32)
    z1d = lax.dot_general(w1b_ref[...], d.reshape(_C, _K * _R),
                          (((1,), (0,)), ((), ())),
                          preferred_element_type=jnp.float32)
    return z1d.reshape(_O1, _K, _R) + z1c[:, None, :]


def _lrelu(v):
    return jnp.where(v >= 0, v, 0.2 * v)


_CNT = float(_B * _N * _K)


def _mlp_body(xrow_ref, d_ref, w1a_ref, w1b_ref, g1_ref, be1_ref, w2_ref,
              g2_ref, be2_ref, out_ref, s1_ref, q1_ref, s2_ref, q2_ref,
              ab1_ref, ab2_ref):
    p = pl.program_id(0)
    first = (pl.program_id(1) == 0) & (pl.program_id(2) == 0)

    def accum(val2d, s_ref, q_ref):
        @pl.when(first)
        def _():
            s_ref[...] = jnp.zeros_like(s_ref)
            q_ref[...] = jnp.zeros_like(q_ref)
        s_ref[...] += jnp.sum(val2d, axis=1)[None, :]
        q_ref[...] += jnp.sum(val2d * val2d, axis=1)[None, :]

    def finalize(s_ref, q_ref, g_ref, be_ref, ab_ref):
        m = s_ref[...] / _CNT
        v = q_ref[...] / _CNT - m * m
        a = g_ref[...] * lax.rsqrt(v + _EPS)
        ab_ref[0:1, :] = a
        ab_ref[1:2, :] = be_ref[...] - a * m

    def z2_of():
        z1 = _z1_of(xrow_ref, d_ref, w1a_ref, w1b_ref)
        a1 = ab1_ref[0]
        b1 = ab1_ref[1]
        y1 = _lrelu(z1 * a1[:, None, None] + b1[:, None, None])
        return lax.dot_general(w2_ref[...], y1.reshape(_O1, _K * _R),
                               (((1,), (0,)), ((), ())),
                               preferred_element_type=jnp.float32)

    @pl.when(p == 0)
    def _():
        z1 = _z1_of(xrow_ref, d_ref, w1a_ref, w1b_ref)
        accum(z1.reshape(_O1, _K * _R), s1_ref, q1_ref)

    @pl.when(p == 1)
    def _():
        @pl.when(first)
        def _():
            finalize(s1_ref, q1_ref, g1_ref, be1_ref, ab1_ref)
        accum(z2_of(), s2_ref, q2_ref)

    @pl.when(p == 2)
    def _():
        @pl.when(first)
        def _():
            finalize(s2_ref, q2_ref, g2_ref, be2_ref, ab2_ref)
        z2 = z2_of()
        a2 = ab2_ref[0]
        b2 = ab2_ref[1]
        y2 = _lrelu(z2 * a2[:, None] + b2[:, None]).reshape(_O2, _K, _R)
        m = y2[:, 0, :]
        for kk in range(1, _K):
            m = jnp.maximum(m, y2[:, kk, :])
        out_ref[0] = m


def _mlp(x, d5, w1a, w1b, g1, be1, w2, g2, be2, interpret=False):
    vec = pl.BlockSpec((1, _O1), lambda p, b, nb: (0, 0))
    wspec = pl.BlockSpec((_O1, _C), lambda p, b, nb: (0, 0))
    return pl.pallas_call(
        _mlp_body,
        grid=(3, _B, _N // _R),
        in_specs=[
            pl.BlockSpec((1, _C, _R), lambda p, b, nb: (b, 0, nb)),
            pl.BlockSpec((1, 1, _C, _K, _R),
                         lambda p, b, nb: (b, nb // 2, 0, 0, nb % 2)),
            wspec, wspec, vec, vec,
            pl.BlockSpec((_O2, _O1), lambda p, b, nb: (0, 0)),
            vec, vec,
        ],
        out_specs=pl.BlockSpec((1, _O2, _R), lambda p, b, nb: (b, 0, nb)),
        out_shape=jax.ShapeDtypeStruct((_B, _O2, _N), jnp.float32),
        scratch_shapes=[
            pltpu.VMEM((1, _O1), jnp.float32),
            pltpu.VMEM((1, _O1), jnp.float32),
            pltpu.VMEM((1, _O2), jnp.float32),
            pltpu.VMEM((1, _O2), jnp.float32),
            pltpu.VMEM((2, _O1), jnp.float32),
            pltpu.VMEM((2, _O2), jnp.float32),
        ],
        interpret=interpret,
    )(x, d5, w1a, w1b, g1, be1, w2, g2, be2)


def kernel(x, W1, gamma1, beta1, W2, gamma2, beta2):
    idx = _knn(x)
    d5 = _gather(x, idx)
    return _mlp(x, d5, W1[:, :_C], W1[:, _C:], gamma1[None, :],
                beta1[None, :], W2, gamma2[None, :], beta2[None, :])
